# baseline (speedup 1.0000x reference)
import jax
import jax.numpy as jnp
from jax import lax
from jax.experimental import pallas as pl
from jax.experimental.pallas import tpu as pltpu
from jax.experimental.pallas import tpu_sc as plsc

B = 16384
F_CAT = 26
NUM_CAT = 100000
TAB_ROWS = NUM_CAT + 1
F_CONT = 13
DIM = 64
F_TOT = F_CAT + F_CONT

NC, NS, L = 2, 16, 16
NW = NC * NS
BPW = B // NW

G = 128
NG_F = BPW // G

CB_C = 512
N_CHUNK_C = BPW * F_CONT // CB_C
RPW_C = BPW * F_CONT


def _tokenizer_body(xcatt_hbm, xc_hbm, tab_hbm, emb_hbm, odcont_hbm,
                    out_hbm,
                    rows_v, idx_v, dst_idx, xc_v, odcont_v, emb_v, sem):
    cid = lax.axis_index("c")
    sid = lax.axis_index("s")
    wid = sid * NC + cid
    base_b = wid * BPW

    pltpu.sync_copy(odcont_hbm, odcont_v)
    pltpu.sync_copy(emb_hbm, emb_v)

    def cat_field(f, carry):
        for k in range(NG_F):
            pltpu.sync_copy(xcatt_hbm.at[pl.ds(f * B + base_b + k * G, G)],
                            idx_v.at[k])
        d_off = base_b * F_TOT + f

        def dst_grp(k, c2):
            for u in range(G // L):
                s = k * G + u * L
                step = lax.broadcasted_iota(jnp.int32, (L,), 0) + s
                dst_idx[k, pl.ds(u * L, L)] = step * F_TOT + d_off
            return c2

        lax.fori_loop(0, NG_F, dst_grp, 0)

        gathers = [
            pltpu.async_copy(tab_hbm.at[f].at[idx_v.at[k]],
                             rows_v.at[pl.ds(k * G, G)], sem)
            for k in range(NG_F)
        ]
        for h in gathers:
            h.wait()
        scatters = [
            pltpu.async_copy(rows_v.at[pl.ds(k * G, G)],
                             out_hbm.at[dst_idx.at[k]], sem)
            for k in range(NG_F)
        ]
        for h in scatters:
            h.wait()
        return carry

    lax.fori_loop(0, F_CAT, cat_field, 0)

    def cont_chunk(it, carry):
        r0 = it * CB_C
        pltpu.sync_copy(xc_hbm.at[pl.ds(base_b * F_CONT + r0, CB_C)], xc_v)
        d_off = base_b * F_TOT

        def grp_body(g, c2):
            s = g * L
            v16 = xc_v[pl.ds(s, L)]
            for lane in range(L):
                rl = s + lane
                f = lax.rem(r0 + rl, F_CONT)
                spl = jnp.full((L,), v16[lane], jnp.float32)
                for q in range(DIM // L):
                    rows_v[rl, pl.ds(q * L, L)] = (
                        spl * emb_v[pl.ds(f * DIM + q * L, L)])
            return c2

        lax.fori_loop(0, CB_C // L, grp_body, 0)

        def dst_grp(k, c2):
            for u in range(G // L):
                s = k * G + u * L
                dst_idx[k, pl.ds(u * L, L)] = (
                    odcont_v[pl.ds(r0 + s, L)] + d_off)
            return c2

        lax.fori_loop(0, CB_C // G, dst_grp, 0)

        scatters = [
            pltpu.async_copy(rows_v.at[pl.ds(k * G, G)],
                             out_hbm.at[dst_idx.at[k]], sem)
            for k in range(CB_C // G)
        ]
        for h in scatters:
            h.wait()
        return carry

    lax.fori_loop(0, N_CHUNK_C, cont_chunk, 0)


_mesh = plsc.VectorSubcoreMesh(core_axis_name="c", subcore_axis_name="s",
                               num_cores=NC, num_subcores=NS)

_sc_call = pl.kernel(
    _tokenizer_body,
    out_type=jax.ShapeDtypeStruct((B * F_TOT, DIM), jnp.float32),
    mesh=_mesh,
    compiler_params=pltpu.CompilerParams(use_tc_tiling_on_sc=False),
    scratch_types=[
        pltpu.VMEM((BPW, DIM), jnp.float32),
        pltpu.VMEM((NG_F, G), jnp.int32),
        pltpu.VMEM((NG_F, G), jnp.int32),
        pltpu.VMEM((CB_C,), jnp.float32),
        pltpu.VMEM((RPW_C,), jnp.int32),
        pltpu.VMEM((F_CONT * DIM,), jnp.float32),
        pltpu.SemaphoreType.DMA,
    ],
)


def kernel(x_categ, x_cont, cat_tables, cont_embeds):
    xcatt = x_categ.T.reshape(B * F_CAT)
    xc = x_cont.reshape(B * F_CONT)
    emb = cont_embeds.reshape(F_CONT * DIM)
    r = jnp.arange(RPW_C, dtype=jnp.int32)
    odcont = (r // F_CONT) * F_TOT + F_CAT + (r % F_CONT)
    out = _sc_call(xcatt, xc, cat_tables, emb, odcont)
    return out.reshape(B, F_TOT, DIM)


# speedup vs baseline: 2.7931x; 2.7931x over previous
import jax
import jax.numpy as jnp
from jax import lax
from jax.experimental import pallas as pl
from jax.experimental.pallas import tpu as pltpu
from jax.experimental.pallas import tpu_sc as plsc

B = 16384
F_CAT = 26
NUM_CAT = 100000
TAB_ROWS = NUM_CAT + 1
F_CONT = 13
DIM = 64
F_TOT = F_CAT + F_CONT

NC, NS, L = 2, 16, 16
NW = NC * NS
BPW = B // NW

G = 128
NG_F = BPW // G

CB_C = 512
N_CHUNK_C = BPW * F_CONT // CB_C
RPW_C = BPW * F_CONT


def _tokenizer_body(*refs):
    tabs = refs[:F_CAT]
    xcatt_hbm, xc_hbm, emb_hbm, odcont_hbm, out_hbm = refs[F_CAT:F_CAT + 5]
    rows_v, idx_v, dst_idx, xc_v, odcont_v, emb_v, sem = refs[F_CAT + 5:]

    cid = lax.axis_index("c")
    sid = lax.axis_index("s")
    wid = sid * NC + cid
    base_b = wid * BPW

    pltpu.sync_copy(odcont_hbm, odcont_v)
    pltpu.sync_copy(emb_hbm, emb_v)

    for f in range(F_CAT):
        for k in range(NG_F):
            pltpu.sync_copy(xcatt_hbm.at[pl.ds(f * B + base_b + k * G, G)],
                            idx_v.at[k])
        d_off = base_b * F_TOT + f

        def dst_grp(k, c2):
            for u in range(G // L):
                s = k * G + u * L
                step = lax.broadcasted_iota(jnp.int32, (L,), 0) + s
                dst_idx[k, pl.ds(u * L, L)] = step * F_TOT + d_off
            return c2

        lax.fori_loop(0, NG_F, dst_grp, 0)

        gathers = [
            pltpu.async_copy(tabs[f].at[idx_v.at[k]],
                             rows_v.at[pl.ds(k * G, G)], sem)
            for k in range(NG_F)
        ]
        for h in gathers:
            h.wait()
        scatters = [
            pltpu.async_copy(rows_v.at[pl.ds(k * G, G)],
                             out_hbm.at[dst_idx.at[k]], sem)
            for k in range(NG_F)
        ]
        for h in scatters:
            h.wait()

    def cont_chunk(it, carry):
        r0 = it * CB_C
        pltpu.sync_copy(xc_hbm.at[pl.ds(base_b * F_CONT + r0, CB_C)], xc_v)
        d_off = base_b * F_TOT

        def grp_body(g, c2):
            s = g * L
            v16 = xc_v[pl.ds(s, L)]
            for lane in range(L):
                rl = s + lane
                f = lax.rem(r0 + rl, F_CONT)
                spl = jnp.full((L,), v16[lane], jnp.float32)
                for q in range(DIM // L):
                    rows_v[rl, pl.ds(q * L, L)] = (
                        spl * emb_v[pl.ds(f * DIM + q * L, L)])
            return c2

        lax.fori_loop(0, CB_C // L, grp_body, 0)

        def dst_grp(k, c2):
            for u in range(G // L):
                s = k * G + u * L
                dst_idx[k, pl.ds(u * L, L)] = (
                    odcont_v[pl.ds(r0 + s, L)] + d_off)
            return c2

        lax.fori_loop(0, CB_C // G, dst_grp, 0)

        scatters = [
            pltpu.async_copy(rows_v.at[pl.ds(k * G, G)],
                             out_hbm.at[dst_idx.at[k]], sem)
            for h, k in zip(range(CB_C // G), range(CB_C // G))
        ]
        for h in scatters:
            h.wait()
        return carry

    lax.fori_loop(0, N_CHUNK_C, cont_chunk, 0)


_mesh = plsc.VectorSubcoreMesh(core_axis_name="c", subcore_axis_name="s",
                               num_cores=NC, num_subcores=NS)

_sc_call = pl.kernel(
    _tokenizer_body,
    out_type=jax.ShapeDtypeStruct((B * F_TOT, DIM), jnp.float32),
    mesh=_mesh,
    compiler_params=pltpu.CompilerParams(use_tc_tiling_on_sc=False),
    scratch_types=[
        pltpu.VMEM((BPW, DIM), jnp.float32),
        pltpu.VMEM((NG_F, G), jnp.int32),
        pltpu.VMEM((NG_F, G), jnp.int32),
        pltpu.VMEM((CB_C,), jnp.float32),
        pltpu.VMEM((RPW_C,), jnp.int32),
        pltpu.VMEM((F_CONT * DIM,), jnp.float32),
        pltpu.SemaphoreType.DMA,
    ],
)


def kernel(x_categ, x_cont, cat_tables, cont_embeds):
    xcatt = x_categ.T.reshape(B * F_CAT)
    xc = x_cont.reshape(B * F_CONT)
    emb = cont_embeds.reshape(F_CONT * DIM)
    r = jnp.arange(RPW_C, dtype=jnp.int32)
    odcont = (r // F_CONT) * F_TOT + F_CAT + (r % F_CONT)
    tabs = [cat_tables[f] for f in range(F_CAT)]
    out = _sc_call(*tabs, xcatt, xc, emb, odcont)
    return out.reshape(B, F_TOT, DIM)


# speedup vs baseline: 2.8459x; 1.0189x over previous
import jax
import jax.numpy as jnp
from jax import lax
from jax.experimental import pallas as pl
from jax.experimental.pallas import tpu as pltpu
from jax.experimental.pallas import tpu_sc as plsc

B = 16384
F_CAT = 26
NUM_CAT = 100000
TAB_ROWS = NUM_CAT + 1
F_CONT = 13
DIM = 64
F_TOT = F_CAT + F_CONT

NC, NS, L = 2, 16, 16
NW = NC * NS
BPW = B // NW

G = 128
NG_F = BPW // G

CB_C = 512
N_CHUNK_C = BPW * F_CONT // CB_C
RPW_C = BPW * F_CONT


def _tokenizer_body(*refs):
    tabs = refs[:F_CAT]
    xcatt_hbm, xc_hbm, emb_hbm, odcont_hbm, out_hbm = refs[F_CAT:F_CAT + 5]
    (rows0, rows1, idx_v, dst0, dst1, bpat_v, xc_v, odcont_v, emb_v,
     gsem, ssem0, ssem1) = refs[F_CAT + 5:]
    rows = (rows0, rows1)
    dsts = (dst0, dst1)
    ssems = (ssem0, ssem1)

    cid = lax.axis_index("c")
    sid = lax.axis_index("s")
    wid = sid * NC + cid
    base_b = wid * BPW

    pltpu.sync_copy(odcont_hbm, odcont_v)
    pltpu.sync_copy(emb_hbm, emb_v)

    def bpat_grp(k, c2):
        for u in range(G // L):
            s = k * G + u * L
            step = lax.broadcasted_iota(jnp.int32, (L,), 0) + (s + base_b)
            bpat_v[k, pl.ds(u * L, L)] = step * F_TOT
        return c2

    lax.fori_loop(0, NG_F, bpat_grp, 0)

    pending = [None, None]
    for f in range(F_CAT):
        bi = f & 1
        rows_v, dst_idx, ssem = rows[bi], dsts[bi], ssems[bi]
        pltpu.sync_copy(xcatt_hbm.at[pl.ds(f * (NW * NG_F) + wid * NG_F,
                                           NG_F)], idx_v)

        def dst_grp(k, c2, _dst=dst_idx, _f=f):
            for u in range(G // L):
                su = pl.ds(u * L, L)
                _dst[k, su] = bpat_v[k, su] + _f
            return c2

        lax.fori_loop(0, NG_F, dst_grp, 0)

        if pending[bi] is not None:
            for h in pending[bi]:
                h.wait()
        gathers = [
            pltpu.async_copy(tabs[f].at[idx_v.at[k]],
                             rows_v.at[pl.ds(k * G, G)], gsem)
            for k in range(NG_F)
        ]
        for h in gathers:
            h.wait()
        pending[bi] = [
            pltpu.async_copy(rows_v.at[pl.ds(k * G, G)],
                             out_hbm.at[dst_idx.at[k]], ssem)
            for k in range(NG_F)
        ]
    for p in pending:
        if p is not None:
            for h in p:
                h.wait()

    rows_v, dst_idx, ssem = rows[0], dsts[0], ssems[0]

    def cont_chunk(it, carry):
        r0 = it * CB_C
        pltpu.sync_copy(xc_hbm.at[pl.ds(base_b * F_CONT + r0, CB_C)], xc_v)
        d_off = base_b * F_TOT

        def grp_body(g, c2):
            s = g * L
            v16 = xc_v[pl.ds(s, L)]
            for lane in range(L):
                rl = s + lane
                f = lax.rem(r0 + rl, F_CONT)
                spl = jnp.full((L,), v16[lane], jnp.float32)
                for q in range(DIM // L):
                    rows_v[rl, pl.ds(q * L, L)] = (
                        spl * emb_v[pl.ds(f * DIM + q * L, L)])
            return c2

        lax.fori_loop(0, CB_C // L, grp_body, 0)

        def dst_grp(k, c2):
            for u in range(G // L):
                s = k * G + u * L
                dst_idx[k, pl.ds(u * L, L)] = (
                    odcont_v[pl.ds(r0 + s, L)] + d_off)
            return c2

        lax.fori_loop(0, CB_C // G, dst_grp, 0)

        scatters = [
            pltpu.async_copy(rows_v.at[pl.ds(k * G, G)],
                             out_hbm.at[dst_idx.at[k]], ssem)
            for k in range(CB_C // G)
        ]
        for h in scatters:
            h.wait()
        return carry

    lax.fori_loop(0, N_CHUNK_C, cont_chunk, 0)


_mesh = plsc.VectorSubcoreMesh(core_axis_name="c", subcore_axis_name="s",
                               num_cores=NC, num_subcores=NS)

_sc_call = pl.kernel(
    _tokenizer_body,
    out_type=jax.ShapeDtypeStruct((B * F_TOT, DIM), jnp.float32),
    mesh=_mesh,
    compiler_params=pltpu.CompilerParams(use_tc_tiling_on_sc=False),
    scratch_types=[
        pltpu.VMEM((BPW, DIM), jnp.float32),
        pltpu.VMEM((BPW, DIM), jnp.float32),
        pltpu.VMEM((NG_F, G), jnp.int32),
        pltpu.VMEM((NG_F, G), jnp.int32),
        pltpu.VMEM((NG_F, G), jnp.int32),
        pltpu.VMEM((NG_F, G), jnp.int32),
        pltpu.VMEM((CB_C,), jnp.float32),
        pltpu.VMEM((RPW_C,), jnp.int32),
        pltpu.VMEM((F_CONT * DIM,), jnp.float32),
        pltpu.SemaphoreType.DMA,
        pltpu.SemaphoreType.DMA,
        pltpu.SemaphoreType.DMA,
    ],
)


def kernel(x_categ, x_cont, cat_tables, cont_embeds):
    xcatt = x_categ.T.reshape(F_CAT * NW * NG_F, G)
    xc = x_cont.reshape(B * F_CONT)
    emb = cont_embeds.reshape(F_CONT * DIM)
    r = jnp.arange(RPW_C, dtype=jnp.int32)
    odcont = (r // F_CONT) * F_TOT + F_CAT + (r % F_CONT)
    tabs = [cat_tables[f] for f in range(F_CAT)]
    out = _sc_call(*tabs, xcatt, xc, emb, odcont)
    return out.reshape(B, F_TOT, DIM)


# speedup vs baseline: 4.0841x; 1.4351x over previous
import jax
import jax.numpy as jnp
from jax import lax
from jax.experimental import pallas as pl
from jax.experimental.pallas import tpu as pltpu
from jax.experimental.pallas import tpu_sc as plsc

B = 16384
F_CAT = 26
NUM_CAT = 100000
TAB_ROWS = NUM_CAT + 1
F_CONT = 13
DIM = 64
F_TOT = F_CAT + F_CONT

NC, NS, L = 2, 16, 16
NW = NC * NS
BPW = B // NW
CB = 8
NCHUNK = BPW // CB
NIDX = CB * F_CAT
NXC = CB * F_CONT


def _tokenizer_body(xcat_hbm, xc_hbm, tab_hbm, emb_hbm, out_hbm,
                    idx_v, xcv, bv0, bv1, slab3, emb_v,
                    gsem0, gsem1, wsem):
    cid = lax.axis_index("c")
    sid = lax.axis_index("s")
    wid = sid * NC + cid
    base_b = wid * BPW

    pltpu.sync_copy(emb_hbm, emb_v)
    bvs = (bv0, bv1)
    gsems = (gsem0, gsem1)

    def drain_g(t):
        pltpu.make_async_copy(tab_hbm.at[pl.ds(0, F_CAT), pl.ds(0, 8), :],
                              bvs[t % 2], gsems[t % 2]).wait()

    def drain_w():
        pltpu.make_async_copy(out_hbm.at[pl.ds(0, CB)], slab3, wsem).wait()

    def wr_all(b0):
        for t in range(CB):
            pltpu.async_copy(slab3.at[t], out_hbm.at[b0 + t], wsem)

    wr_all(base_b)

    def chunk_body(c):
        b0 = base_b + c * CB
        pltpu.sync_copy(xcat_hbm.at[pl.ds(b0 * F_CAT, NIDX)],
                        idx_v.at[pl.ds(0, NIDX)])
        pltpu.sync_copy(xc_hbm.at[pl.ds(b0 * F_CONT, NXC)],
                        xcv.at[pl.ds(0, NXC)])

        gcache = {}

        def cat_scalar(j):
            g = j // L
            if g not in gcache:
                gcache[g] = idx_v[pl.ds(g * L, L)]
            return gcache[g][j % L]

        def fire_t(t):
            bv, gs = bvs[t % 2], gsems[t % 2]
            for f in range(F_CAT):
                s = cat_scalar(t * F_CAT + f)
                s8 = pl.multiple_of((s // 8) * 8, 8)
                pltpu.async_copy(tab_hbm.at[f, pl.ds(s8, 8), :],
                                 bv.at[f], gs)

        fire_t(0)
        drain_w()

        def cgrp(g, c2):
            s = g * L
            v16 = xcv[pl.ds(s, L)]
            for lane in range(L):
                rl = s + lane
                t = rl // F_CONT
                j = lax.rem(rl, F_CONT)
                spl = jnp.full((L,), v16[lane], jnp.float32)
                for q in range(DIM // L):
                    slab3[t, F_CAT + j, pl.ds(q * L, L)] = (
                        spl * emb_v[pl.ds(j * DIM + q * L, L)])
            return c2

        lax.fori_loop(0, NXC // L, cgrp, 0)
        v16 = xcv[pl.ds((NXC // L) * L, L)]
        for lane in range(NXC % L):
            rl = (NXC // L) * L + lane
            t, j = rl // F_CONT, rl % F_CONT
            spl = jnp.full((L,), v16[lane], jnp.float32)
            for q in range(DIM // L):
                slab3[t, F_CAT + j, pl.ds(q * L, L)] = (
                    spl * emb_v[pl.ds(j * DIM + q * L, L)])

        for t in range(CB):
            if t + 1 < CB:
                fire_t(t + 1)
            drain_g(t)
            bv = bvs[t % 2]
            for f in range(F_CAT):
                r = lax.rem(cat_scalar(t * F_CAT + f), 8)
                for q in range(DIM // L):
                    slab3[t, f, pl.ds(q * L, L)] = bv[f, r, pl.ds(q * L, L)]
        wr_all(b0)

    def chunk(c, carry):
        chunk_body(c)
        return carry

    lax.fori_loop(0, NCHUNK, chunk, 0)
    drain_w()


_mesh = plsc.VectorSubcoreMesh(core_axis_name="c", subcore_axis_name="s",
                               num_cores=NC, num_subcores=NS)

_sc_call = pl.kernel(
    _tokenizer_body,
    out_type=jax.ShapeDtypeStruct((B, F_TOT, DIM), jnp.float32),
    mesh=_mesh,
    scratch_types=[
        pltpu.VMEM((((NIDX + L - 1) // L) * L,), jnp.int32),
        pltpu.VMEM((((NXC + L - 1) // L) * L,), jnp.float32),
        pltpu.VMEM((F_CAT, 8, DIM), jnp.float32),
        pltpu.VMEM((F_CAT, 8, DIM), jnp.float32),
        pltpu.VMEM((CB, F_TOT, DIM), jnp.float32),
        pltpu.VMEM((F_CONT * DIM,), jnp.float32),
        pltpu.SemaphoreType.DMA,
        pltpu.SemaphoreType.DMA,
        pltpu.SemaphoreType.DMA,
    ],
)


def kernel(x_categ, x_cont, cat_tables, cont_embeds):
    xcat = x_categ.reshape(B * F_CAT)
    xc = x_cont.reshape(B * F_CONT)
    emb = cont_embeds.reshape(F_CONT * DIM)
    return _sc_call(xcat, xc, cat_tables, emb)


# speedup vs baseline: 4.1061x; 1.0054x over previous
import jax
import jax.numpy as jnp
from jax import lax
from jax.experimental import pallas as pl
from jax.experimental.pallas import tpu as pltpu
from jax.experimental.pallas import tpu_sc as plsc

B = 16384
F_CAT = 26
NUM_CAT = 100000
TAB_ROWS = NUM_CAT + 1
F_CONT = 13
DIM = 64
F_TOT = F_CAT + F_CONT

NC, NS, L = 2, 16, 16
NW = NC * NS
BPW = B // NW
CB = 8
NCHUNK = BPW // CB
NIDX = CB * F_CAT
NXC = CB * F_CONT
NXCP = 112
IPW = BPW * F_CAT
XPW = NCHUNK * NXCP


def _tokenizer_body(xcat_hbm, xcp_hbm, tab_hbm, emb_hbm, out_hbm,
                    idx_all, xc_all, bv0, bv1, slab3, emb_v,
                    gsem0, gsem1, wsem):
    cid = lax.axis_index("c")
    sid = lax.axis_index("s")
    wid = sid * NC + cid
    base_b = wid * BPW

    pltpu.sync_copy(emb_hbm, emb_v)
    pltpu.sync_copy(xcat_hbm.at[pl.ds(wid * IPW, IPW)], idx_all)
    pltpu.sync_copy(xcp_hbm.at[pl.ds(wid * XPW, XPW)], xc_all)
    bvs = (bv0, bv1)
    gsems = (gsem0, gsem1)

    def drain_g(t):
        pltpu.make_async_copy(tab_hbm.at[pl.ds(0, F_CAT), pl.ds(0, 8), :],
                              bvs[t % 2], gsems[t % 2]).wait()

    def drain_w():
        pltpu.make_async_copy(out_hbm.at[pl.ds(0, CB)], slab3, wsem).wait()

    def wr_all(b0):
        for t in range(CB):
            pltpu.async_copy(slab3.at[t], out_hbm.at[b0 + t], wsem)

    wr_all(base_b)

    def chunk_body(c):
        b0 = base_b + c * CB
        i0 = c * NIDX
        x0 = c * NXCP

        gcache = {}

        def cat_scalar(j):
            g = j // L
            if g not in gcache:
                gcache[g] = idx_all[pl.ds(i0 + g * L, L)]
            return gcache[g][j % L]

        def fire_t(t):
            bv, gs = bvs[t % 2], gsems[t % 2]
            for f in range(F_CAT):
                s = cat_scalar(t * F_CAT + f)
                s8 = pl.multiple_of((s // 8) * 8, 8)
                pltpu.async_copy(tab_hbm.at[f, pl.ds(s8, 8), :],
                                 bv.at[f], gs)

        fire_t(0)
        drain_w()

        def cgrp(g, c2):
            s = g * L
            v16 = xc_all[pl.ds(x0 + s, L)]
            for lane in range(L):
                rl = s + lane
                t = rl // F_CONT
                j = lax.rem(rl, F_CONT)
                spl = jnp.full((L,), v16[lane], jnp.float32)
                for q in range(DIM // L):
                    slab3[t, F_CAT + j, pl.ds(q * L, L)] = (
                        spl * emb_v[pl.ds(j * DIM + q * L, L)])
            return c2

        lax.fori_loop(0, NXC // L, cgrp, 0)
        v16 = xc_all[pl.ds(x0 + (NXC // L) * L, L)]
        for lane in range(NXC % L):
            rl = (NXC // L) * L + lane
            t, j = rl // F_CONT, rl % F_CONT
            spl = jnp.full((L,), v16[lane], jnp.float32)
            for q in range(DIM // L):
                slab3[t, F_CAT + j, pl.ds(q * L, L)] = (
                    spl * emb_v[pl.ds(j * DIM + q * L, L)])

        for t in range(CB):
            if t + 1 < CB:
                fire_t(t + 1)
            drain_g(t)
            bv = bvs[t % 2]
            for f in range(F_CAT):
                r = lax.rem(cat_scalar(t * F_CAT + f), 8)
                for q in range(DIM // L):
                    slab3[t, f, pl.ds(q * L, L)] = bv[f, r, pl.ds(q * L, L)]
        wr_all(b0)

    def chunk(c, carry):
        chunk_body(c)
        return carry

    lax.fori_loop(0, NCHUNK, chunk, 0)
    drain_w()


_mesh = plsc.VectorSubcoreMesh(core_axis_name="c", subcore_axis_name="s",
                               num_cores=NC, num_subcores=NS)

_sc_call = pl.kernel(
    _tokenizer_body,
    out_type=jax.ShapeDtypeStruct((B, F_TOT, DIM), jnp.float32),
    mesh=_mesh,
    scratch_types=[
        pltpu.VMEM((IPW,), jnp.int32),
        pltpu.VMEM((XPW,), jnp.float32),
        pltpu.VMEM((F_CAT, 8, DIM), jnp.float32),
        pltpu.VMEM((F_CAT, 8, DIM), jnp.float32),
        pltpu.VMEM((CB, F_TOT, DIM), jnp.float32),
        pltpu.VMEM((F_CONT * DIM,), jnp.float32),
        pltpu.SemaphoreType.DMA,
        pltpu.SemaphoreType.DMA,
        pltpu.SemaphoreType.DMA,
    ],
)


def kernel(x_categ, x_cont, cat_tables, cont_embeds):
    xcat = x_categ.reshape(B * F_CAT)
    xcp = jnp.pad(x_cont.reshape(B // CB, NXC),
                  ((0, 0), (0, NXCP - NXC))).reshape(-1)
    emb = cont_embeds.reshape(F_CONT * DIM)
    return _sc_call(xcat, xcp, cat_tables, emb)


# speedup vs baseline: 4.1376x; 1.0076x over previous
import jax
import jax.numpy as jnp
from jax import lax
from jax.experimental import pallas as pl
from jax.experimental.pallas import tpu as pltpu
from jax.experimental.pallas import tpu_sc as plsc

B = 16384
F_CAT = 26
NUM_CAT = 100000
TAB_ROWS = NUM_CAT + 1
F_CONT = 13
DIM = 64
F_TOT = F_CAT + F_CONT

NC, NS, L = 2, 16, 16
NW = NC * NS
BPW = B // NW
CB = 8
NCHUNK = BPW // CB
NIDX = CB * F_CAT
NXC = CB * F_CONT
DEPTH = 3


def _tokenizer_body(xcat_hbm, xc_hbm, tab_hbm, emb_hbm, out_hbm,
                    idx_v, xcv, bv0, bv1, bv2, slab3, emb_v,
                    gsem0, gsem1, gsem2, wsem):
    cid = lax.axis_index("c")
    sid = lax.axis_index("s")
    wid = sid * NC + cid
    base_b = wid * BPW

    pltpu.sync_copy(emb_hbm, emb_v)
    bvs = (bv0, bv1, bv2)
    gsems = (gsem0, gsem1, gsem2)

    def drain_g(t):
        pltpu.make_async_copy(tab_hbm.at[pl.ds(0, F_CAT), pl.ds(0, 8), :],
                              bvs[t % DEPTH], gsems[t % DEPTH]).wait()

    def drain_w():
        pltpu.make_async_copy(out_hbm.at[pl.ds(0, CB)], slab3, wsem).wait()

    def wr_all(b0):
        for t in range(CB):
            pltpu.async_copy(slab3.at[t], out_hbm.at[b0 + t], wsem)

    wr_all(base_b)

    def chunk_body(c):
        b0 = base_b + c * CB
        pltpu.sync_copy(xcat_hbm.at[pl.ds(b0 * F_CAT, NIDX)],
                        idx_v.at[pl.ds(0, NIDX)])
        pltpu.sync_copy(xc_hbm.at[pl.ds(b0 * F_CONT, NXC)],
                        xcv.at[pl.ds(0, NXC)])

        gcache = {}

        def cat_scalar(j):
            g = j // L
            if g not in gcache:
                gcache[g] = idx_v[pl.ds(g * L, L)]
            return gcache[g][j % L]

        def fire_t(t):
            bv, gs = bvs[t % DEPTH], gsems[t % DEPTH]
            for f in range(F_CAT):
                s = cat_scalar(t * F_CAT + f)
                s8 = pl.multiple_of((s // 8) * 8, 8)
                pltpu.async_copy(tab_hbm.at[f, pl.ds(s8, 8), :],
                                 bv.at[f], gs)

        fire_t(0)
        fire_t(1)
        drain_w()

        def cgrp(g, c2):
            s = g * L
            v16 = xcv[pl.ds(s, L)]
            for lane in range(L):
                rl = s + lane
                t = rl // F_CONT
                j = lax.rem(rl, F_CONT)
                spl = jnp.full((L,), v16[lane], jnp.float32)
                for q in range(DIM // L):
                    slab3[t, F_CAT + j, pl.ds(q * L, L)] = (
                        spl * emb_v[pl.ds(j * DIM + q * L, L)])
            return c2

        lax.fori_loop(0, NXC // L, cgrp, 0)
        v16 = xcv[pl.ds((NXC // L) * L, L)]
        for lane in range(NXC % L):
            rl = (NXC // L) * L + lane
            t, j = rl // F_CONT, rl % F_CONT
            spl = jnp.full((L,), v16[lane], jnp.float32)
            for q in range(DIM // L):
                slab3[t, F_CAT + j, pl.ds(q * L, L)] = (
                    spl * emb_v[pl.ds(j * DIM + q * L, L)])

        for t in range(CB):
            if t + 2 < CB:
                fire_t(t + 2)
            drain_g(t)
            bv = bvs[t % DEPTH]
            for f in range(F_CAT):
                r = lax.rem(cat_scalar(t * F_CAT + f), 8)
                for q in range(DIM // L):
                    slab3[t, f, pl.ds(q * L, L)] = bv[f, r, pl.ds(q * L, L)]
        wr_all(b0)

    def chunk(c, carry):
        chunk_body(c)
        return carry

    lax.fori_loop(0, NCHUNK, chunk, 0)
    drain_w()


_mesh = plsc.VectorSubcoreMesh(core_axis_name="c", subcore_axis_name="s",
                               num_cores=NC, num_subcores=NS)

_sc_call = pl.kernel(
    _tokenizer_body,
    out_type=jax.ShapeDtypeStruct((B, F_TOT, DIM), jnp.float32),
    mesh=_mesh,
    scratch_types=[
        pltpu.VMEM((((NIDX + L - 1) // L) * L,), jnp.int32),
        pltpu.VMEM((((NXC + L - 1) // L) * L,), jnp.float32),
        pltpu.VMEM((F_CAT, 8, DIM), jnp.float32),
        pltpu.VMEM((F_CAT, 8, DIM), jnp.float32),
        pltpu.VMEM((F_CAT, 8, DIM), jnp.float32),
        pltpu.VMEM((CB, F_TOT, DIM), jnp.float32),
        pltpu.VMEM((F_CONT * DIM,), jnp.float32),
        pltpu.SemaphoreType.DMA,
        pltpu.SemaphoreType.DMA,
        pltpu.SemaphoreType.DMA,
        pltpu.SemaphoreType.DMA,
    ],
)


def kernel(x_categ, x_cont, cat_tables, cont_embeds):
    xcat = x_categ.reshape(B * F_CAT)
    xc = x_cont.reshape(B * F_CONT)
    emb = cont_embeds.reshape(F_CONT * DIM)
    return _sc_call(xcat, xc, cat_tables, emb)
